# baseline (speedup 1.0000x reference)
import functools

import jax
import jax.numpy as jnp
from jax import lax
from jax.experimental import pallas as pl
from jax.experimental.pallas import tpu as pltpu
from jax.experimental.pallas import tpu_sc as plsc


def _sc_row_copy(src, n_rows):
    dim = src.shape[1]
    info = plsc.get_sparse_core_info()
    nw = info.num_cores * info.num_subcores
    rows_per_w = n_rows // nw
    chunk = 128
    n_chunks = rows_per_w // chunk
    assert n_rows % nw == 0 and rows_per_w % chunk == 0

    mesh = plsc.VectorSubcoreMesh(core_axis_name="c", subcore_axis_name="s")

    @functools.partial(
        pl.kernel,
        mesh=mesh,
        out_type=jax.ShapeDtypeStruct((n_rows, dim), src.dtype),
        scratch_types=[
            pltpu.VMEM((chunk, dim), src.dtype),
            pltpu.VMEM((chunk, dim), src.dtype),
            pltpu.SemaphoreType.DMA,
            pltpu.SemaphoreType.DMA,
            pltpu.SemaphoreType.DMA,
            pltpu.SemaphoreType.DMA,
        ],
    )
    def body(src_hbm, dst_hbm, buf0, buf1, si0, si1, so0, so1):
        bufs, sin, sout = (buf0, buf1), (si0, si1), (so0, so1)
        wid = lax.axis_index("s") * info.num_cores + lax.axis_index("c")
        base = wid * rows_per_w

        def in_copy(i):
            return pltpu.make_async_copy(
                src_hbm.at[pl.ds(base + i * chunk, chunk)], bufs[i % 2], sin[i % 2])

        def out_copy(i):
            return pltpu.make_async_copy(
                bufs[i % 2], dst_hbm.at[pl.ds(base + i * chunk, chunk)], sout[i % 2])

        in_copy(0).start()
        for i in range(n_chunks):
            if i + 1 < n_chunks:
                if i >= 1:
                    out_copy(i - 1).wait()
                in_copy(i + 1).start()
            in_copy(i).wait()
            out_copy(i).start()
        if n_chunks >= 2:
            out_copy(n_chunks - 2).wait()
        out_copy(n_chunks - 1).wait()

    return body(src[:n_rows])


def _tc_dma_copy(src, n_rows, n_chunks=8):
    dim = src.shape[1]
    rows_per = n_rows // n_chunks
    assert n_rows % n_chunks == 0

    def body(src_ref, dst_ref, *sems):
        copies = [
            pltpu.make_async_copy(
                src_ref.at[pl.ds(i * rows_per, rows_per)],
                dst_ref.at[pl.ds(i * rows_per, rows_per)],
                sems[i])
            for i in range(n_chunks)
        ]
        for c in copies:
            c.start()
        for c in copies:
            c.wait()

    return pl.pallas_call(
        body,
        in_specs=[pl.BlockSpec(memory_space=pl.ANY)],
        out_specs=pl.BlockSpec(memory_space=pl.ANY),
        out_shape=jax.ShapeDtypeStruct((n_rows, dim), src.dtype),
        scratch_shapes=[pltpu.SemaphoreType.DMA] * n_chunks,
    )(src[:n_rows])


def kernel(item, out):
    max_length = out.shape[0]
    b = item.shape[0]
    if max_length == 0:
        return item
    if b < max_length:
        return _tc_dma_copy(item, b)
    filled = _sc_row_copy(item, max_length)
    tail = item[max_length:]
    if tail.shape[0]:
        filled = jax.lax.dynamic_update_slice(filled, tail, (0, 0))
    return filled


# speedup vs baseline: 16.4586x; 16.4586x over previous
import functools

import jax
import jax.numpy as jnp
from jax import lax
from jax.experimental import pallas as pl
from jax.experimental.pallas import tpu as pltpu
from jax.experimental.pallas import tpu_sc as plsc


def _sc_row_copy(src, n_rows, chunk=64, nbuf=4):
    dim = src.shape[1]
    info = plsc.get_sparse_core_info()
    nw = info.num_cores * info.num_subcores
    rows_per_w = n_rows // nw
    n_chunks = rows_per_w // chunk
    assert n_rows % nw == 0 and rows_per_w % chunk == 0
    nbuf = min(nbuf, n_chunks)

    mesh = plsc.VectorSubcoreMesh(core_axis_name="c", subcore_axis_name="s")

    @functools.partial(
        pl.kernel,
        mesh=mesh,
        out_type=jax.ShapeDtypeStruct((n_rows, dim), src.dtype),
        scratch_types=(
            [pltpu.VMEM((chunk, dim), src.dtype)] * nbuf
            + [pltpu.SemaphoreType.DMA] * (2 * nbuf)
        ),
    )
    def body(src_hbm, dst_hbm, *scratch):
        bufs = scratch[:nbuf]
        sin = scratch[nbuf:2 * nbuf]
        sout = scratch[2 * nbuf:]
        wid = lax.axis_index("s") * info.num_cores + lax.axis_index("c")
        base = wid * rows_per_w

        def in_copy(i):
            return pltpu.make_async_copy(
                src_hbm.at[pl.ds(base + i * chunk, chunk)],
                bufs[i % nbuf], sin[i % nbuf])

        def out_copy(i):
            return pltpu.make_async_copy(
                bufs[i % nbuf],
                dst_hbm.at[pl.ds(base + i * chunk, chunk)], sout[i % nbuf])

        for j in range(nbuf):
            in_copy(j).start()
        for i in range(n_chunks):
            in_copy(i).wait()
            out_copy(i).start()
            if i + nbuf < n_chunks:
                out_copy(i).wait()
                in_copy(i + nbuf).start()
        for i in range(max(0, n_chunks - nbuf), n_chunks):
            out_copy(i).wait()

    return body(src[:n_rows])


def _tc_dma_copy(src, n_rows, n_chunks=8):
    dim = src.shape[1]
    rows_per = n_rows // n_chunks
    assert n_rows % n_chunks == 0

    def body(src_ref, dst_ref, *sems):
        copies = [
            pltpu.make_async_copy(
                src_ref.at[pl.ds(i * rows_per, rows_per)],
                dst_ref.at[pl.ds(i * rows_per, rows_per)],
                sems[i])
            for i in range(n_chunks)
        ]
        for c in copies:
            c.start()
        for c in copies:
            c.wait()

    return pl.pallas_call(
        body,
        in_specs=[pl.BlockSpec(memory_space=pl.ANY)],
        out_specs=pl.BlockSpec(memory_space=pl.ANY),
        out_shape=jax.ShapeDtypeStruct((n_rows, dim), src.dtype),
        scratch_shapes=[pltpu.SemaphoreType.DMA] * n_chunks,
    )(src[:n_rows])


def kernel(item, out):
    max_length = out.shape[0]
    b = item.shape[0]
    if max_length == 0:
        return item
    if b < max_length:
        return _sc_row_copy(item, b)
    filled = _sc_row_copy(item, max_length)
    tail = item[max_length:]
    if tail.shape[0]:
        filled = jax.lax.dynamic_update_slice(filled, tail, (0, 0))
    return filled


# speedup vs baseline: 16.6190x; 1.0097x over previous
import functools

import jax
import jax.numpy as jnp
from jax import lax
from jax.experimental import pallas as pl
from jax.experimental.pallas import tpu as pltpu
from jax.experimental.pallas import tpu_sc as plsc


def _sc_row_copy(src, n_rows, chunk=32, nbuf=14):
    dim = src.shape[1]
    info = plsc.get_sparse_core_info()
    nw = info.num_cores * info.num_subcores
    rows_per_w = n_rows // nw
    n_chunks = rows_per_w // chunk
    assert n_rows % nw == 0 and rows_per_w % chunk == 0
    nbuf = min(nbuf, n_chunks)

    mesh = plsc.VectorSubcoreMesh(core_axis_name="c", subcore_axis_name="s")

    @functools.partial(
        pl.kernel,
        mesh=mesh,
        out_type=jax.ShapeDtypeStruct((n_rows, dim), src.dtype),
        scratch_types=(
            [pltpu.VMEM((chunk, dim), src.dtype)] * nbuf
            + [pltpu.SemaphoreType.DMA] * (2 * nbuf)
        ),
    )
    def body(src_hbm, dst_hbm, *scratch):
        bufs = scratch[:nbuf]
        sin = scratch[nbuf:2 * nbuf]
        sout = scratch[2 * nbuf:]
        wid = lax.axis_index("s") * info.num_cores + lax.axis_index("c")
        base = wid * rows_per_w

        def in_copy(i):
            return pltpu.make_async_copy(
                src_hbm.at[pl.ds(base + i * chunk, chunk)],
                bufs[i % nbuf], sin[i % nbuf])

        def out_copy(i):
            return pltpu.make_async_copy(
                bufs[i % nbuf],
                dst_hbm.at[pl.ds(base + i * chunk, chunk)], sout[i % nbuf])

        for j in range(nbuf):
            in_copy(j).start()
        for i in range(n_chunks):
            in_copy(i).wait()
            out_copy(i).start()
            if i + nbuf < n_chunks:
                out_copy(i).wait()
                in_copy(i + nbuf).start()
        for i in range(max(0, n_chunks - nbuf), n_chunks):
            out_copy(i).wait()

    return body(src[:n_rows])


def _tc_dma_copy(src, n_rows, n_chunks=8):
    dim = src.shape[1]
    rows_per = n_rows // n_chunks
    assert n_rows % n_chunks == 0

    def body(src_ref, dst_ref, *sems):
        copies = [
            pltpu.make_async_copy(
                src_ref.at[pl.ds(i * rows_per, rows_per)],
                dst_ref.at[pl.ds(i * rows_per, rows_per)],
                sems[i])
            for i in range(n_chunks)
        ]
        for c in copies:
            c.start()
        for c in copies:
            c.wait()

    return pl.pallas_call(
        body,
        in_specs=[pl.BlockSpec(memory_space=pl.ANY)],
        out_specs=pl.BlockSpec(memory_space=pl.ANY),
        out_shape=jax.ShapeDtypeStruct((n_rows, dim), src.dtype),
        scratch_shapes=[pltpu.SemaphoreType.DMA] * n_chunks,
    )(src[:n_rows])


def kernel(item, out):
    max_length = out.shape[0]
    b = item.shape[0]
    if max_length == 0:
        return item
    if b < max_length:
        return _sc_row_copy(item, b)
    filled = _sc_row_copy(item, max_length)
    tail = item[max_length:]
    if tail.shape[0]:
        filled = jax.lax.dynamic_update_slice(filled, tail, (0, 0))
    return filled
